# baseline (speedup 1.0000x reference)
import jax
import jax.numpy as jnp
from jax import lax
from jax.experimental import pallas as pl
from jax.experimental.pallas import tpu as pltpu

_TOP_K = 96
_GATHER_CHUNK = 16


_SCORE_STREAMS = 8
_SCORE_ROWS = 8


def _score_body(*refs):
    xs = refs[:-1]
    o_ref = refs[-1]
    rb = _SCORE_ROWS
    for j, x_ref in enumerate(xs):
        o_ref[0, 0, j * rb : (j + 1) * rb] = jnp.max(
            jnp.abs(x_ref[...]), axis=(1, 2)
        )


def _topk_body(k: int, s_ref, o_ref):
    s = s_ref[...]
    b, c = s.shape
    si = s[:, :, None]
    sj = s[:, None, :]
    ii = lax.broadcasted_iota(jnp.int32, (b, c, c), 1)
    jj = lax.broadcasted_iota(jnp.int32, (b, c, c), 2)
    beats = (sj > si) | ((sj == si) & (jj < ii))
    rank = jnp.sum(beats.astype(jnp.int32), axis=2)
    pos = lax.broadcasted_iota(jnp.int32, (b, c, k), 2)
    chan = lax.broadcasted_iota(jnp.int32, (b, c, k), 1)
    hit = rank[:, :, None] == pos
    o_ref[...] = jnp.sum(jnp.where(hit, chan, 0), axis=1)


def _gather_body(idx_ref, *refs):
    del idx_ref
    xs = refs[:-1]
    o_ref = refs[-1]
    for j, x_ref in enumerate(xs):
        o_ref[0, j] = x_ref[0, 0]


def _channel_topk_pool(x, k: int):
    b, c, h, w = x.shape
    rows = b * c
    x3 = x.reshape(rows, h, w)

    ns, rb = _SCORE_STREAMS, _SCORE_ROWS
    step_rows = ns * rb
    grid_n = rows // step_rows

    def _score_in_spec(j):
        return pl.BlockSpec((rb, h, w), lambda i: (i * ns + j, 0, 0))

    scores2 = pl.pallas_call(
        _score_body,
        grid=(grid_n,),
        in_specs=[_score_in_spec(j) for j in range(ns)],
        out_specs=pl.BlockSpec((1, 8, step_rows), lambda i: (i, 0, 0)),
        out_shape=jax.ShapeDtypeStruct((grid_n, 8, step_rows), jnp.float32),
        compiler_params=pltpu.CompilerParams(
            dimension_semantics=("parallel",)
        ),
    )(*([x3] * ns))
    scores = scores2[:, 0, :].reshape(b, c)

    idx = pl.pallas_call(
        lambda s_ref, o_ref: _topk_body(k, s_ref, o_ref),
        in_specs=[pl.BlockSpec((b, c), lambda: (0, 0))],
        out_specs=pl.BlockSpec((b, k), lambda: (0, 0)),
        out_shape=jax.ShapeDtypeStruct((b, k), jnp.int32),
    )(scores)

    g = _GATHER_CHUNK

    def _in_spec(j):
        return pl.BlockSpec(
            (1, 1, h, w),
            lambda bi, ki, idx_r: (bi, idx_r[bi, ki * g + j], 0, 0),
        )

    out = pl.pallas_call(
        _gather_body,
        grid_spec=pltpu.PrefetchScalarGridSpec(
            num_scalar_prefetch=1,
            grid=(b, k // g),
            in_specs=[_in_spec(j) for j in range(g)],
            out_specs=pl.BlockSpec(
                (1, g, h, w), lambda bi, ki, idx_r: (bi, ki, 0, 0)
            ),
        ),
        out_shape=jax.ShapeDtypeStruct((b, k, h, w), jnp.float32),
        compiler_params=pltpu.CompilerParams(
            dimension_semantics=("parallel", "parallel")
        ),
    )(idx, *([x] * g))
    return out


def kernel(x):
    return _channel_topk_pool(x, _TOP_K)
